# baseline (speedup 1.0000x reference)
import functools

import jax
import jax.numpy as jnp
from jax import lax
from jax.experimental import pallas as pl
from jax.experimental.pallas import tpu as pltpu
from jax.experimental.pallas import tpu_sc as plsc

A = 2000
V = 50
D = 128
B = 4096
NJ = V - 1
NBUF = 6


def kernel(batch_attributes, batch_values, proto_bank):
    table = proto_bank.transpose(1, 0, 2).reshape(V * A, D)

    info = plsc.get_sparse_core_info()
    NC, NS, L = info.num_cores, info.num_subcores, info.num_lanes
    NW = NC * NS
    S = B // NW

    mesh = plsc.VectorSubcoreMesh(core_axis_name="c", subcore_axis_name="s")

    @functools.partial(
        pl.kernel,
        mesh=mesh,
        out_type=(
            jax.ShapeDtypeStruct((B, D), jnp.float32),
            jax.ShapeDtypeStruct((NJ * B, D), jnp.float32),
        ),
        scratch_types=[
            pltpu.VMEM((S,), jnp.int32),
            pltpu.VMEM((S,), jnp.int32),
            pltpu.VMEM((S,), jnp.int32),
            pltpu.VMEM((NJ, S), jnp.int32),
            pltpu.VMEM((S, D), jnp.float32),
            pltpu.VMEM((NBUF, S, D), jnp.float32),
            pltpu.SemaphoreType.DMA,
            pltpu.SemaphoreType.DMA,
            pltpu.SemaphoreType.DMA,
            pltpu.SemaphoreType.DMA,
            pltpu.SemaphoreType.DMA,
            pltpu.SemaphoreType.DMA,
            pltpu.SemaphoreType.DMA,
            pltpu.SemaphoreType.DMA,
            pltpu.SemaphoreType.DMA,
            pltpu.SemaphoreType.DMA,
            pltpu.SemaphoreType.DMA,
            pltpu.SemaphoreType.DMA,
            pltpu.SemaphoreType.DMA,
        ],
    )
    def sc_kernel(attr_hbm, val_hbm, table_hbm, pos_hbm, neg_hbm,
                  attr_v, val_v, pidx_v, nidx_v, posbuf, ring,
                  psem, g0s, g1s, g2s, g3s, g4s, g5s,
                  d0s, d1s, d2s, d3s, d4s, d5s):
        gsems = (g0s, g1s, g2s, g3s, g4s, g5s)
        dsems = (d0s, d1s, d2s, d3s, d4s, d5s)
        wid = lax.axis_index("s") * NC + lax.axis_index("c")
        base = wid * S
        pltpu.sync_copy(attr_hbm.at[pl.ds(base, S)], attr_v)
        pltpu.sync_copy(val_hbm.at[pl.ds(base, S)], val_v)

        def compute_planes(lo, hi):
            def jloop(j, carry):
                for c in range(S // L):
                    s0 = c * L
                    a = attr_v[pl.ds(s0, L)]
                    v = val_v[pl.ds(s0, L)]
                    nidx_v[j, pl.ds(s0, L)] = (
                        j * A + a + jnp.where(v <= j, A, 0))
                return carry

            lax.fori_loop(lo, hi, jloop, 0)

        def gather(j, b):
            pltpu.async_copy(table_hbm.at[nidx_v.at[j]], ring.at[b], gsems[b])

        def wait_gather(j, b):
            pltpu.make_async_copy(
                table_hbm.at[nidx_v.at[j]], ring.at[b], gsems[b]).wait()

        def drain(j, b):
            pltpu.async_copy(
                ring.at[b], neg_hbm.at[pl.ds(j * B + base, S)], dsems[b])

        def wait_drain(j, b):
            pltpu.make_async_copy(
                ring.at[b], neg_hbm.at[pl.ds(j * B + base, S)],
                dsems[b]).wait()

        compute_planes(0, NBUF - 2)
        for b in range(NBUF - 2):
            gather(b, b)

        for c in range(S // L):
            s0 = c * L
            a = attr_v[pl.ds(s0, L)]
            v = val_v[pl.ds(s0, L)]
            pidx_v[pl.ds(s0, L)] = v * A + a
        pos_copy = pltpu.async_copy(table_hbm.at[pidx_v], posbuf, psem)
        compute_planes(NBUF - 2, NJ)

        def step(r, carry):
            for k in range(NBUF):
                b = k
                j = NBUF * r + k
                bn = (k + NBUF - 2) % NBUF
                wait_gather(j, b)
                drain(j, b)
                if k <= 1:
                    @pl.when(r >= 1)
                    def _():
                        wait_drain(j - 2, bn)

                    gather(j + NBUF - 2, bn)
                else:
                    wait_drain(j - 2, bn)

                    @pl.when(j + NBUF - 2 < NJ)
                    def _():
                        gather(j + NBUF - 2, bn)
            return carry

        lax.fori_loop(0, NJ // NBUF, step, 0)

        wait_drain(NJ - 3, (NJ - 3) % NBUF)
        wait_drain(NJ - 2, (NJ - 2) % NBUF)
        wait_gather(NJ - 1, (NJ - 1) % NBUF)
        drain(NJ - 1, (NJ - 1) % NBUF)
        pos_copy.wait()
        pltpu.sync_copy(posbuf, pos_hbm.at[pl.ds(base, S)])
        wait_drain(NJ - 1, (NJ - 1) % NBUF)

    pos, neg = sc_kernel(batch_attributes, batch_values, table)
    return pos, neg.reshape(NJ, B, D).transpose(1, 0, 2)
